# baseline (speedup 1.0000x reference)
import functools

import jax
import jax.numpy as jnp
from jax.experimental import pallas as pl
from jax.experimental.pallas import tpu as pltpu

_E = 8
_G = 8


def _moe_body(x_ref, wg_ref, W1_ref, b1_ref, W2_ref, b2_ref,
              y_ref, loss_ref, imp_ref, load_ref, *, nb):
    step = pl.program_id(0)
    T = x_ref.shape[1]
    x2 = x_ref[...].reshape(_G * T, x_ref.shape[2])
    rows = jax.lax.broadcasted_iota(jnp.int32, (_G, _G * T), 0)
    cols = jax.lax.broadcasted_iota(jnp.int32, (_G, _G * T), 1)
    sel = (cols // T == rows).astype(jnp.float32) * (1.0 / T)
    xg = jnp.dot(sel, x2, preferred_element_type=jnp.float32)
    logits = jnp.dot(xg, wg_ref[...],
                     preferred_element_type=jnp.float32)
    logits = jnp.clip(logits, -50.0, 50.0)
    iota = jax.lax.broadcasted_iota(jnp.int32, (_G, _E), 1)
    v0 = jnp.max(logits, axis=1, keepdims=True)
    e0 = jnp.min(jnp.where(logits == v0, iota, _E), axis=1, keepdims=True)
    masked = jnp.where(iota == e0, -jnp.inf, logits)
    v1 = jnp.max(masked, axis=1, keepdims=True)
    e1 = jnp.min(jnp.where(masked == v1, iota, _E), axis=1, keepdims=True)
    t = jnp.exp(v1 - v0)
    g0 = 1.0 / (1.0 + t)
    g1 = t / (1.0 + t)

    gates = (jnp.where(iota == e0, g0, 0.0)
             + jnp.where(iota == e1, g1, 0.0))

    @pl.when(step == 0)
    def _init():
        imp_ref[...] = jnp.zeros_like(imp_ref)
        load_ref[...] = jnp.zeros_like(load_ref)

    imp_ref[...] += jnp.sum(gates, axis=0, keepdims=True)
    load_ref[...] += jnp.sum((gates > 0.0).astype(jnp.float32),
                             axis=0, keepdims=True)

    def half_expert(xi, e, g):
        h = jnp.dot(xi, W1_ref[e], preferred_element_type=jnp.float32)
        h = h + b1_ref[e][None, :]
        return (0.5 * g) * (h + h * jax.lax.erf(h * 0.7071067811865476))

    for i in range(_G):
        li = logits[i:i + 1]
        ii = iota[i:i + 1]
        v0i = jnp.max(li)
        e0i = jnp.min(jnp.where(li == v0i, ii, _E))
        mi = jnp.where(ii == e0i, -jnp.inf, li)
        v1i = jnp.max(mi)
        e1i = jnp.min(jnp.where(mi == v1i, ii, _E))
        ti = jnp.exp(v1i - v0i)
        g0i = 1.0 / (1.0 + ti)
        g1i = ti / (1.0 + ti)
        xi = x_ref[i]
        h0 = half_expert(xi, e0i, g0i)
        h1 = half_expert(xi, e1i, g1i)
        o = (jnp.dot(h0, W2_ref[e0i], preferred_element_type=jnp.float32)
             + jnp.dot(h1, W2_ref[e1i], preferred_element_type=jnp.float32))
        bw = g0i * b2_ref[e0i][None, :] + g1i * b2_ref[e1i][None, :]
        y_ref[i] = o + bw

    @pl.when(step == nb - 1)
    def _loss():
        n = float(_E)
        eps = 1e-10

        def cv_sq(v):
            m = jnp.sum(v) / n
            var = jnp.sum((v - m) ** 2) / (n - 1.0)
            return var / (m * m + eps)

        loss = cv_sq(imp_ref[...]) + cv_sq(load_ref[...])
        loss_ref[...] = jnp.clip(loss, 0.0, 1000.0) * jnp.ones((1, 1),
                                                               jnp.float32)


def kernel(x, w_gate, W1, b1, W2, b2):
    B, H, W, C = x.shape
    T = H * W
    E = w_gate.shape[1]
    x_flat = x.reshape(B, T, C)
    nb = B // _G

    y_flat, loss = pl.pallas_call(
        functools.partial(_moe_body, nb=nb),
        grid=(nb,),
        in_specs=[
            pl.BlockSpec((_G, T, C), lambda b: (b, 0, 0)),
            pl.BlockSpec((C, E), lambda b: (0, 0)),
            pl.BlockSpec(W1.shape, lambda b: (0, 0, 0)),
            pl.BlockSpec(b1.shape, lambda b: (0, 0)),
            pl.BlockSpec(W2.shape, lambda b: (0, 0, 0)),
            pl.BlockSpec(b2.shape, lambda b: (0, 0)),
        ],
        out_specs=[
            pl.BlockSpec((_G, T, C), lambda b: (b, 0, 0)),
            pl.BlockSpec((1, 1), lambda b: (0, 0)),
        ],
        out_shape=[
            jax.ShapeDtypeStruct((B, T, C), jnp.float32),
            jax.ShapeDtypeStruct((1, 1), jnp.float32),
        ],
        scratch_shapes=[
            pltpu.VMEM((1, _E), jnp.float32),
            pltpu.VMEM((1, _E), jnp.float32),
        ],
    )(x_flat, w_gate, W1, b1, W2, b2)

    return y_flat.reshape(B, H, W, C), loss[0, 0]


# speedup vs baseline: 1.0093x; 1.0093x over previous
import functools

import jax
import jax.numpy as jnp
from jax.experimental import pallas as pl
from jax.experimental.pallas import tpu as pltpu

_E = 8
_G = 8


def _moe_body(x_ref, wg_ref, W1_ref, b1_ref, W2_ref, b2_ref,
              y_ref, loss_ref, imp_ref, load_ref, *, nb):
    step = pl.program_id(0)
    xg = jnp.mean(x_ref[...], axis=1)
    logits = jnp.dot(xg, wg_ref[...],
                     preferred_element_type=jnp.float32)
    logits = jnp.clip(logits, -50.0, 50.0)
    iota = jax.lax.broadcasted_iota(jnp.int32, (_G, _E), 1)
    v0 = jnp.max(logits, axis=1, keepdims=True)
    e0 = jnp.min(jnp.where(logits == v0, iota, _E), axis=1, keepdims=True)
    masked = jnp.where(iota == e0, -jnp.inf, logits)
    v1 = jnp.max(masked, axis=1, keepdims=True)
    e1 = jnp.min(jnp.where(masked == v1, iota, _E), axis=1, keepdims=True)
    t = jnp.exp(v1 - v0)
    g0 = 1.0 / (1.0 + t)
    g1 = t / (1.0 + t)

    gates = (jnp.where(iota == e0, g0, 0.0)
             + jnp.where(iota == e1, g1, 0.0))

    @pl.when(step == 0)
    def _init():
        imp_ref[...] = jnp.zeros_like(imp_ref)
        load_ref[...] = jnp.zeros_like(load_ref)

    imp_ref[...] += jnp.sum(gates, axis=0, keepdims=True)
    load_ref[...] += jnp.sum((gates > 0.0).astype(jnp.float32),
                             axis=0, keepdims=True)

    def half_expert(xi, e, g):
        h = jnp.dot(xi, W1_ref[e], preferred_element_type=jnp.float32)
        h = h + b1_ref[e][None, :]
        return (0.5 * g) * (h + h * jax.lax.erf(h * 0.7071067811865476))

    for i in range(_G):
        li = logits[i:i + 1]
        ii = iota[i:i + 1]
        v0i = jnp.max(li)
        e0i = jnp.min(jnp.where(li == v0i, ii, _E))
        mi = jnp.where(ii == e0i, -jnp.inf, li)
        v1i = jnp.max(mi)
        e1i = jnp.min(jnp.where(mi == v1i, ii, _E))
        ti = jnp.exp(v1i - v0i)
        g0i = 1.0 / (1.0 + ti)
        g1i = ti / (1.0 + ti)
        xi = x_ref[i]
        h0 = half_expert(xi, e0i, g0i)
        h1 = half_expert(xi, e1i, g1i)
        o = (jnp.dot(h0, W2_ref[e0i], preferred_element_type=jnp.float32)
             + jnp.dot(h1, W2_ref[e1i], preferred_element_type=jnp.float32))
        bw = g0i * b2_ref[e0i][None, :] + g1i * b2_ref[e1i][None, :]
        y_ref[i] = o + bw

    @pl.when(step == nb - 1)
    def _loss():
        n = float(_E)
        eps = 1e-10

        def cv_sq(v):
            m = jnp.sum(v) / n
            var = jnp.sum((v - m) ** 2) / (n - 1.0)
            return var / (m * m + eps)

        loss = cv_sq(imp_ref[...]) + cv_sq(load_ref[...])
        loss_ref[...] = jnp.clip(loss, 0.0, 1000.0) * jnp.ones((1, 1),
                                                               jnp.float32)


def kernel(x, w_gate, W1, b1, W2, b2):
    B, H, W, C = x.shape
    T = H * W
    E = w_gate.shape[1]
    x_flat = x.reshape(B, T, C)
    nb = B // _G

    y_flat, loss = pl.pallas_call(
        functools.partial(_moe_body, nb=nb),
        grid=(nb,),
        in_specs=[
            pl.BlockSpec((_G, T, C), lambda b: (b, 0, 0)),
            pl.BlockSpec((C, E), lambda b: (0, 0)),
            pl.BlockSpec(W1.shape, lambda b: (0, 0, 0)),
            pl.BlockSpec(b1.shape, lambda b: (0, 0)),
            pl.BlockSpec(W2.shape, lambda b: (0, 0, 0)),
            pl.BlockSpec(b2.shape, lambda b: (0, 0)),
        ],
        out_specs=[
            pl.BlockSpec((_G, T, C), lambda b: (b, 0, 0)),
            pl.BlockSpec((1, 1), lambda b: (0, 0)),
        ],
        out_shape=[
            jax.ShapeDtypeStruct((B, T, C), jnp.float32),
            jax.ShapeDtypeStruct((1, 1), jnp.float32),
        ],
        scratch_shapes=[
            pltpu.VMEM((1, _E), jnp.float32),
            pltpu.VMEM((1, _E), jnp.float32),
        ],
    )(x_flat, w_gate, W1, b1, W2, b2)

    return y_flat.reshape(B, H, W, C), loss[0, 0]


# speedup vs baseline: 1.0407x; 1.0311x over previous
import functools

import jax
import jax.numpy as jnp
from jax.experimental import pallas as pl
from jax.experimental.pallas import tpu as pltpu

_E = 8
_G = 8


def _moe_body(x_ref, wg_ref, W1_ref, b1_ref, W2_ref, b2_ref,
              y_ref, loss_ref, imp_ref, load_ref, *, nb):
    step = pl.program_id(0)
    xg = jnp.mean(x_ref[...], axis=1)
    logits = jnp.dot(xg, wg_ref[...],
                     preferred_element_type=jnp.float32)
    logits = jnp.clip(logits, -50.0, 50.0)
    iota = jax.lax.broadcasted_iota(jnp.int32, (_G, _E), 1)
    v0 = jnp.max(logits, axis=1, keepdims=True)
    e0 = jnp.min(jnp.where(logits == v0, iota, _E), axis=1, keepdims=True)
    masked = jnp.where(iota == e0, -jnp.inf, logits)
    v1 = jnp.max(masked, axis=1, keepdims=True)
    e1 = jnp.min(jnp.where(masked == v1, iota, _E), axis=1, keepdims=True)
    t = jnp.exp(v1 - v0)
    g0 = 1.0 / (1.0 + t)
    g1 = t / (1.0 + t)

    gates = (jnp.where(iota == e0, g0, 0.0)
             + jnp.where(iota == e1, g1, 0.0))

    @pl.when(step == 0)
    def _init():
        imp_ref[...] = jnp.zeros_like(imp_ref)
        load_ref[...] = jnp.zeros_like(load_ref)

    imp_ref[...] += jnp.sum(gates, axis=0, keepdims=True)
    load_ref[...] += jnp.sum((gates > 0.0).astype(jnp.float32),
                             axis=0, keepdims=True)

    def expert(xi, e):
        h = jnp.dot(xi, W1_ref[e], preferred_element_type=jnp.float32)
        h = h + b1_ref[e][None, :]
        h = 0.5 * h * (1.0 + jax.lax.erf(h * 0.7071067811865476))
        o = jnp.dot(h, W2_ref[e], preferred_element_type=jnp.float32)
        return o + b2_ref[e][None, :]

    for i in range(_G):
        li = logits[i:i + 1]
        ii = iota[i:i + 1]
        v0i = jnp.max(li)
        e0i = jnp.min(jnp.where(li == v0i, ii, _E))
        mi = jnp.where(ii == e0i, -jnp.inf, li)
        v1i = jnp.max(mi)
        e1i = jnp.min(jnp.where(mi == v1i, ii, _E))
        ti = jnp.exp(v1i - v0i)
        g0i = 1.0 / (1.0 + ti)
        g1i = ti / (1.0 + ti)
        xi = x_ref[i]
        y_ref[i] = g0i * expert(xi, e0i) + g1i * expert(xi, e1i)

    @pl.when(step == nb - 1)
    def _loss():
        n = float(_E)
        eps = 1e-10

        def cv_sq(v):
            m = jnp.sum(v) / n
            var = jnp.sum((v - m) ** 2) / (n - 1.0)
            return var / (m * m + eps)

        loss = cv_sq(imp_ref[...]) + cv_sq(load_ref[...])
        loss_ref[...] = jnp.clip(loss, 0.0, 1000.0) * jnp.ones((1, 1),
                                                               jnp.float32)


def kernel(x, w_gate, W1, b1, W2, b2):
    B, H, W, C = x.shape
    T = H * W
    E = w_gate.shape[1]
    x_flat = x.reshape(B, T, C)
    nb = B // _G

    y_flat, loss = pl.pallas_call(
        functools.partial(_moe_body, nb=nb),
        grid=(nb,),
        in_specs=[
            pl.BlockSpec((_G, T, C), lambda b: (b, 0, 0)),
            pl.BlockSpec((C, E), lambda b: (0, 0)),
            pl.BlockSpec(W1.shape, lambda b: (0, 0, 0)),
            pl.BlockSpec(b1.shape, lambda b: (0, 0)),
            pl.BlockSpec(W2.shape, lambda b: (0, 0, 0)),
            pl.BlockSpec(b2.shape, lambda b: (0, 0)),
        ],
        out_specs=[
            pl.BlockSpec((_G, T, C), lambda b: (b, 0, 0)),
            pl.BlockSpec((1, 1), lambda b: (0, 0)),
        ],
        out_shape=[
            jax.ShapeDtypeStruct((B, T, C), jnp.float32),
            jax.ShapeDtypeStruct((1, 1), jnp.float32),
        ],
        scratch_shapes=[
            pltpu.VMEM((1, _E), jnp.float32),
            pltpu.VMEM((1, _E), jnp.float32),
        ],
    )(x_flat, w_gate, W1, b1, W2, b2)

    return y_flat.reshape(B, H, W, C), loss[0, 0]
